# baseline (speedup 1.0000x reference)
import functools

import jax
import jax.numpy as jnp
from jax import lax
from jax.experimental import pallas as pl
from jax.experimental.pallas import tpu as pltpu
from jax.experimental.pallas import tpu_sc as plsc

B = 16384
L = 50
M = 64
N = B * L
NW = 32
CPW = N // NW
CHUNK = 256
NCH = CPW // CHUNK
BB = CHUNK // 128
MH = M // 8

_mesh = plsc.VectorSubcoreMesh(core_axis_name="c", subcore_axis_name="s")


@functools.partial(
    pl.kernel,
    mesh=_mesh,
    out_type=jax.ShapeDtypeStruct((L, MH, B // 128, 8, 128), jnp.float32),
    scratch_types=[
        pltpu.VMEM((CPW,), jnp.int32),
        pltpu.VMEM((2, CHUNK, M), jnp.float32),
        pltpu.VMEM((2, MH, BB, 8, 128), jnp.float32),
        pltpu.SemaphoreType.DMA((2,)),
        pltpu.SemaphoreType.DMA((2,)),
    ],
    compiler_params=pltpu.CompilerParams(
        use_tc_tiling_on_sc=False, needs_layout_passes=False
    ),
)
def _emb_gather(z_hbm, table_hbm, out_hbm, idx_v, rows_v, t_v, gsem, osem):
    w = lax.axis_index("s") * 2 + lax.axis_index("c")
    base = w * CPW
    iota = lax.iota(jnp.int32, 16)

    pltpu.sync_copy(z_hbm.at[pl.ds(base, CPW)], idx_v)

    def start_gather(c, buf):
        pltpu.async_copy(
            table_hbm.at[idx_v.at[pl.ds(c * CHUNK, CHUNK)]],
            rows_v.at[buf],
            gsem.at[buf],
        )

    def wait_gather(buf):
        pltpu.make_async_copy(
            table_hbm.at[idx_v.at[pl.ds(0, CHUNK)]], rows_v.at[buf], gsem.at[buf]
        ).wait()

    def wait_writes(buf):
        pltpu.make_async_copy(
            t_v.at[buf], out_hbm.at[0, :, pl.ds(0, BB)], osem.at[buf]
        ).wait()

    start_gather(0, 0)

    def body(c, carry):
        buf = lax.rem(c, 2)
        nbuf = lax.rem(c + 1, 2)
        s = base + c * CHUNK
        l = s // B
        bb0 = lax.rem(s, B) // 128

        @pl.when(c + 1 < NCH)
        def _():
            start_gather(c + 1, nbuf)

        wait_gather(buf)

        @pl.when(lax.rem(s, B) == 0)
        def _():
            for j in range(M // 16):
                rows_v[buf, 0, pl.ds(j * 16, 16)] = jnp.zeros((16,), jnp.float32)

        @pl.when(c >= 2)
        def _():
            wait_writes(buf)

        bvec = jnp.full((16,), buf, jnp.int32)

        def tbody(u, tc):
            mh = u // BB
            bb = lax.rem(u, BB)
            rvecs = [iota + (bb * 128 + b16 * 16) for b16 in range(8)]
            for ml in range(8):
                cvec = jnp.full((16,), mh * 8 + ml, jnp.int32)
                for b16 in range(8):
                    v = plsc.load_gather(rows_v, [bvec, rvecs[b16], cvec])
                    t_v[buf, mh, bb, ml, pl.ds(b16 * 16, 16)] = v
            return tc

        lax.fori_loop(0, MH * BB, tbody, 0)

        for mh in range(MH):
            pltpu.async_copy(
                t_v.at[buf, mh],
                out_hbm.at[l, mh, pl.ds(bb0, BB)],
                osem.at[buf],
            )
        return carry

    lax.fori_loop(0, NCH, body, 0)
    wait_writes(0)
    wait_writes(1)


def kernel(y, table):
    y32 = y.astype(jnp.int32)
    zt = jnp.concatenate([jnp.zeros((1, L), jnp.int32), y32[: B - 1]], axis=0)
    zt = zt.T.reshape(-1)
    out5 = _emb_gather(zt, table)
    out = out5.transpose(2, 4, 0, 1, 3).reshape(B, L, M)
    return out


# speedup vs baseline: 1.7934x; 1.7934x over previous
import functools

import jax
import jax.numpy as jnp
from jax import lax
from jax.experimental import pallas as pl
from jax.experimental.pallas import tpu as pltpu
from jax.experimental.pallas import tpu_sc as plsc

B = 16384
L = 50
M = 64
N = B * L
NW = 32
CPW = N // NW
CHUNK = 256
NCH = CPW // CHUNK
BB = CHUNK // 128
MH = M // 8
TWORDS = MH * BB * 8 * 128

_mesh = plsc.VectorSubcoreMesh(core_axis_name="c", subcore_axis_name="s")


@functools.partial(
    pl.kernel,
    mesh=_mesh,
    out_type=jax.ShapeDtypeStruct((L, MH, B // 128, 8, 128), jnp.float32),
    scratch_types=[
        pltpu.VMEM((CPW,), jnp.int32),
        pltpu.VMEM((2, CHUNK, M), jnp.float32),
        pltpu.VMEM((2, MH, BB, 8, 128), jnp.float32),
        pltpu.SemaphoreType.DMA((2,)),
        pltpu.SemaphoreType.DMA((2,)),
    ],
    compiler_params=pltpu.CompilerParams(
        use_tc_tiling_on_sc=False, needs_layout_passes=False
    ),
)
def _emb_gather(z_hbm, table_hbm, out_hbm, idx_v, rows_v, t_v, gsem, osem):
    w = lax.axis_index("s") * 2 + lax.axis_index("c")
    base = w * CPW
    iota = lax.iota(jnp.int32, 16)
    rotc = [jnp.bitwise_and(iota + d, 15) for d in range(16)]
    mhc = [q >> 3 for q in rotc]
    mlc = [jnp.bitwise_and(q, 7) for q in rotc]

    pltpu.sync_copy(z_hbm.at[pl.ds(base, CPW)], idx_v)

    def start_gather(c, buf):
        pltpu.async_copy(
            table_hbm.at[idx_v.at[pl.ds(c * CHUNK, CHUNK)]],
            rows_v.at[buf],
            gsem.at[buf],
        )

    def wait_gather(buf):
        pltpu.make_async_copy(
            table_hbm.at[idx_v.at[pl.ds(0, CHUNK)]], rows_v.at[buf], gsem.at[buf]
        ).wait()

    def wait_writes(buf):
        pltpu.make_async_copy(
            t_v.at[buf], out_hbm.at[0, :, pl.ds(0, BB)], osem.at[buf]
        ).wait()

    start_gather(0, 0)

    def body(c, carry):
        buf = lax.rem(c, 2)
        nbuf = lax.rem(c + 1, 2)
        s = base + c * CHUNK
        l = s // B
        bb0 = lax.rem(s, B) // 128

        @pl.when(c + 1 < NCH)
        def _():
            start_gather(c + 1, nbuf)

        wait_gather(buf)

        @pl.when(lax.rem(s, B) == 0)
        def _():
            for j in range(M // 16):
                rows_v[buf, 0, pl.ds(j * 16, 16)] = jnp.zeros((16,), jnp.float32)

        @pl.when(c >= 2)
        def _():
            wait_writes(buf)

        bufvec = jnp.full((16,), buf, jnp.int32)

        def tbody(sb, tc):
            b16 = sb // 4
            cb = lax.rem(sb, 4)
            rowvec = iota + b16 * 16
            bbvec = jnp.full((16,), b16 // 8, jnp.int32)
            blvec = iota + lax.rem(b16, 8) * 16
            mh0 = cb * 2
            c0 = cb * 16
            for d in range(16):
                v = plsc.load_gather(rows_v, [bufvec, rowvec, rotc[d] + c0])
                plsc.store_scatter(
                    t_v, [bufvec, mhc[d] + mh0, bbvec, mlc[d], blvec], v
                )
            return tc

        lax.fori_loop(0, 16 * 4, tbody, 0)

        for mh in range(MH):
            pltpu.async_copy(
                t_v.at[buf, mh],
                out_hbm.at[l, mh, pl.ds(bb0, BB)],
                osem.at[buf],
            )
        return carry

    lax.fori_loop(0, NCH, body, 0)
    wait_writes(0)
    wait_writes(1)


def kernel(y, table):
    y32 = y.astype(jnp.int32)
    zt = jnp.concatenate([jnp.zeros((1, L), jnp.int32), y32[: B - 1]], axis=0)
    zt = zt.T.reshape(-1)
    out5 = _emb_gather(zt, table)
    return out5.transpose(2, 4, 0, 1, 3).reshape(B, L, M)
